# baseline (speedup 1.0000x reference)
import functools

import jax
import jax.numpy as jnp
from jax import lax
from jax.experimental import pallas as pl
from jax.experimental.pallas import tpu as pltpu
from jax.experimental.pallas import tpu_sc as plsc

N = 10000
NP = 10240
NT = 32
NS = 16
EPT = 10000
B = 125
NB = EPT // B
RPT = NP // NS

_MESH = plsc.VectorSubcoreMesh(core_axis_name="c", subcore_axis_name="s")


@functools.partial(
    pl.kernel,
    out_type=jax.ShapeDtypeStruct((2, NP), jnp.float32),
    mesh=_MESH,
    scratch_types=[
        pltpu.VMEM((NB, B), jnp.int32),
        pltpu.VMEM((B,), jnp.float32),
        pltpu.VMEM_SHARED((NP,), jnp.float32),
    ],
)
def _deg(dst_hbm, ones_hbm, zero_hbm, out_hbm, didx, ones_v, acc):
    c = lax.axis_index("c")
    s = lax.axis_index("s")
    wid = c * NS + s
    base = s * RPT

    pltpu.sync_copy(ones_hbm, ones_v)
    pltpu.sync_copy(dst_hbm.at[wid], didx)
    pltpu.sync_copy(zero_hbm.at[pl.ds(base, RPT)], acc.at[pl.ds(base, RPT)])
    plsc.subcore_barrier()

    def body(j, carry):
        pltpu.sync_copy(ones_v, acc.at[didx.at[j]], add=True)
        return carry

    lax.fori_loop(0, NB, body, 0)
    plsc.subcore_barrier()
    pltpu.sync_copy(acc.at[pl.ds(base, RPT)], out_hbm.at[c, pl.ds(base, RPT)])


def _make_agg(D, tc_tiling=True):
    @functools.partial(
        pl.kernel,
        out_type=jax.ShapeDtypeStruct((2, NP, D), jnp.float32),
        mesh=_MESH,
        compiler_params=pltpu.CompilerParams(use_tc_tiling_on_sc=tc_tiling),
        scratch_types=[
            pltpu.VMEM_SHARED((NP, D), jnp.float32),
            pltpu.SemaphoreType.DMA,
            pltpu.SemaphoreType.DMA,
        ],
    )
    def agg(xs_hbm, src_hbm, dst_hbm, zero_hbm, out_hbm, acc, sem0, sem1):
        c = lax.axis_index("c")
        s = lax.axis_index("s")
        wid = c * NS + s
        base = s * RPT

        NH = NB // 2

        def run(sidx, didx, rows0, rows1):
            pltpu.sync_copy(zero_hbm.at[pl.ds(base, RPT)],
                            acc.at[pl.ds(base, RPT)])
            plsc.subcore_barrier()

            for p in range(2):
                pltpu.sync_copy(src_hbm.at[wid, pl.ds(p * NH, NH)], sidx)
                pltpu.sync_copy(dst_hbm.at[wid, pl.ds(p * NH, NH)], didx)
                pltpu.async_copy(xs_hbm.at[sidx.at[0]], rows0, sem0)

                def body(jj, carry):
                    j0 = 2 * jj
                    g1 = pltpu.async_copy(xs_hbm.at[sidx.at[j0 + 1]], rows1,
                                          sem1)
                    pltpu.make_async_copy(xs_hbm.at[sidx.at[j0]], rows0,
                                          sem0).wait()
                    pltpu.sync_copy(rows0, acc.at[didx.at[j0]], add=True)

                    @pl.when(jj < NH // 2 - 1)
                    def _():
                        pltpu.async_copy(xs_hbm.at[sidx.at[j0 + 2]], rows0,
                                         sem0)

                    g1.wait()
                    pltpu.sync_copy(rows1, acc.at[didx.at[j0 + 1]], add=True)
                    return carry

                lax.fori_loop(0, NH // 2, body, 0)

            plsc.subcore_barrier()
            pltpu.sync_copy(acc.at[pl.ds(base, RPT)],
                            out_hbm.at[c, pl.ds(base, RPT)])

        pl.run_scoped(run,
                      pltpu.VMEM((NB // 2, B), jnp.int32),
                      pltpu.VMEM((NB // 2, B), jnp.int32),
                      pltpu.VMEM((B, D), jnp.float32),
                      pltpu.VMEM((B, D), jnp.float32))

    return agg


_agg128 = _make_agg(128)
_agg64 = _make_agg(64, tc_tiling=False)


_RB = 1024


def _dinv_of(degp_ref):
    d = degp_ref[0, :, 0] + degp_ref[1, :, 0] + 1.0
    return lax.rsqrt(d)[:, None]


def _tc_scale_body(degp_ref, x_ref, o_ref):
    o_ref[...] = x_ref[...] * _dinv_of(degp_ref)


def _tc_mid_body(degp_ref, p_ref, xs_ref, w1_ref, b1_ref, w2_ref, o_ref):
    dinv = _dinv_of(degp_ref)
    agg = (p_ref[0] + p_ref[1] + xs_ref[...]) * dinv
    h = jnp.maximum(jnp.dot(agg, w1_ref[...],
                            preferred_element_type=jnp.float32)
                    + b1_ref[...], 0.0)
    o_ref[...] = jnp.dot(h, w2_ref[...],
                         preferred_element_type=jnp.float32) * dinv


def _tc_dec_body(degp_ref, q_ref, xs_ref, b2_ref, w1_ref, b1_ref, w2_ref,
                 b2d_ref, o_ref):
    dinv = _dinv_of(degp_ref)
    z = (q_ref[0] + q_ref[1] + xs_ref[...]) * dinv + b2_ref[...]
    h = jnp.maximum(jnp.dot(z, w1_ref[...],
                            preferred_element_type=jnp.float32)
                    + b1_ref[...], 0.0)
    o_ref[...] = jnp.dot(h, w2_ref[...],
                         preferred_element_type=jnp.float32) + b2d_ref[...]


def _row_spec(d):
    return pl.BlockSpec((_RB, d), lambda i: (i, 0))


def _part_spec(d):
    return pl.BlockSpec((2, _RB, d), lambda i: (0, i, 0))


def _full_spec(a, b):
    return pl.BlockSpec((a, b), lambda i: (0, 0))


def _tc_scale(degp, x):
    return pl.pallas_call(
        _tc_scale_body,
        grid=(NP // _RB,),
        in_specs=[_part_spec(1), _row_spec(128)],
        out_specs=_row_spec(128),
        out_shape=jax.ShapeDtypeStruct((NP, 128), jnp.float32),
    )(degp, x)


def _tc_mid(degp, p, xs1, w1, b1, w2):
    return pl.pallas_call(
        _tc_mid_body,
        grid=(NP // _RB,),
        in_specs=[_part_spec(1), _part_spec(128), _row_spec(128),
                  _full_spec(128, 256), _full_spec(1, 256),
                  _full_spec(256, 64)],
        out_specs=_row_spec(64),
        out_shape=jax.ShapeDtypeStruct((NP, 64), jnp.float32),
    )(degp, p, xs1, w1, b1, w2)


def _tc_dec(degp, q, xs2, b2, w1, b1, w2, b2d):
    return pl.pallas_call(
        _tc_dec_body,
        grid=(NP // _RB,),
        in_specs=[_part_spec(1), _part_spec(64), _row_spec(64),
                  _full_spec(1, 64), _full_spec(64, 256), _full_spec(1, 256),
                  _full_spec(256, 8), _full_spec(1, 8)],
        out_specs=_row_spec(8),
        out_shape=jax.ShapeDtypeStruct((NP, 8), jnp.float32),
    )(degp, q, xs2, b2, w1, b1, w2, b2d)


def kernel(x, edge_index, conv1_W, conv1_b, conv2_W, conv2_b,
           fc1_W, fc1_b, fc2_W, fc2_b):
    ei = edge_index.astype(jnp.int32)
    src = ei[0].reshape(NT, NB, B)
    dst = ei[1].reshape(NT, NB, B)

    xpad = jnp.pad(x, ((0, NP - N), (0, 0)))
    zeros128 = jnp.zeros((NP, 128), jnp.float32)

    degp = _deg(dst, jnp.ones((B,), jnp.float32), zeros128[:, 0])[:, :, None]
    xs1 = _tc_scale(degp, xpad)
    p = _agg128(xs1, src, dst, zeros128)
    xs2 = _tc_mid(degp, p, xs1, conv1_W, conv1_b[None, :], conv2_W)
    q = _agg64(xs2, src, dst, zeros128[:, :64])

    fc2p = jnp.pad(fc2_W[:, :6], ((0, 0), (0, 2)))
    fc2bp = jnp.pad(fc2_b[:6], (0, 2))
    out = _tc_dec(degp, q, xs2, conv2_b[None, :], fc1_W, fc1_b[None, :],
                  fc2p, fc2bp[None, :])
    return out[:N, :6]


# speedup vs baseline: 1.0169x; 1.0169x over previous
import functools

import jax
import jax.numpy as jnp
from jax import lax
from jax.experimental import pallas as pl
from jax.experimental.pallas import tpu as pltpu
from jax.experimental.pallas import tpu_sc as plsc

N = 10000
NP = 10240
NT = 32
NS = 16
EPT = 10000
B = 125
NB = EPT // B
RPT = NP // NS

_MESH = plsc.VectorSubcoreMesh(core_axis_name="c", subcore_axis_name="s")


@functools.partial(
    pl.kernel,
    out_type=jax.ShapeDtypeStruct((2, NP), jnp.float32),
    mesh=_MESH,
    scratch_types=[
        pltpu.VMEM((NB, B), jnp.int32),
        pltpu.VMEM((B,), jnp.float32),
        pltpu.VMEM_SHARED((NP,), jnp.float32),
        pltpu.SemaphoreType.DMA,
    ],
)
def _deg(dst_hbm, ones_hbm, zero_hbm, out_hbm, didx, ones_v, acc, dsem):
    c = lax.axis_index("c")
    s = lax.axis_index("s")
    wid = c * NS + s
    base = s * RPT

    pltpu.sync_copy(ones_hbm, ones_v)
    pltpu.sync_copy(dst_hbm.at[wid], didx)
    pltpu.sync_copy(zero_hbm.at[pl.ds(base, RPT)], acc.at[pl.ds(base, RPT)])
    plsc.subcore_barrier()

    def body(ch, carry):
        def fire(k, c2):
            pltpu.async_copy(ones_v, acc.at[didx.at[ch * 16 + k]], dsem,
                             add=True)
            return c2

        lax.fori_loop(0, 16, fire, 0)

        def drain(k, c2):
            pltpu.make_async_copy(ones_v, acc.at[didx.at[ch * 16 + k]],
                                  dsem).wait()
            return c2

        lax.fori_loop(0, 16, drain, 0)
        return carry

    lax.fori_loop(0, NB // 16, body, 0)
    plsc.subcore_barrier()
    pltpu.sync_copy(acc.at[pl.ds(base, RPT)], out_hbm.at[c, pl.ds(base, RPT)])


def _make_agg(D, tc_tiling=True):
    @functools.partial(
        pl.kernel,
        out_type=jax.ShapeDtypeStruct((2, NP, D), jnp.float32),
        mesh=_MESH,
        compiler_params=pltpu.CompilerParams(use_tc_tiling_on_sc=tc_tiling),
        scratch_types=[
            pltpu.VMEM_SHARED((NP, D), jnp.float32),
            pltpu.SemaphoreType.DMA,
            pltpu.SemaphoreType.DMA,
            pltpu.SemaphoreType.DMA,
            pltpu.SemaphoreType.DMA,
        ],
    )
    def agg(xs_hbm, src_hbm, dst_hbm, zero_hbm, out_hbm, acc, ga0, ga1,
            sc0, sc1):
        c = lax.axis_index("c")
        s = lax.axis_index("s")
        wid = c * NS + s
        base = s * RPT

        NH = NB // 2

        def run(sidx, didx, rows0, rows1):
            pltpu.sync_copy(zero_hbm.at[pl.ds(base, RPT)],
                            acc.at[pl.ds(base, RPT)])
            plsc.subcore_barrier()

            for p in range(2):
                pltpu.sync_copy(src_hbm.at[wid, pl.ds(p * NH, NH)], sidx)
                pltpu.sync_copy(dst_hbm.at[wid, pl.ds(p * NH, NH)], didx)
                pltpu.async_copy(xs_hbm.at[sidx.at[0]], rows0, ga0)

                def body(jj, carry):
                    j0 = 2 * jj

                    @pl.when(jj > 0)
                    def _():
                        pltpu.make_async_copy(
                            rows1, acc.at[didx.at[j0 - 1]], sc1).wait()

                    pltpu.async_copy(xs_hbm.at[sidx.at[j0 + 1]], rows1, ga1)
                    pltpu.make_async_copy(xs_hbm.at[sidx.at[j0]], rows0,
                                          ga0).wait()
                    pltpu.async_copy(rows0, acc.at[didx.at[j0]], sc0,
                                     add=True)

                    @pl.when(jj < NH // 2 - 1)
                    def _():
                        pltpu.make_async_copy(
                            rows0, acc.at[didx.at[j0]], sc0).wait()
                        pltpu.async_copy(xs_hbm.at[sidx.at[j0 + 2]], rows0,
                                         ga0)

                    pltpu.make_async_copy(xs_hbm.at[sidx.at[j0 + 1]], rows1,
                                          ga1).wait()
                    pltpu.async_copy(rows1, acc.at[didx.at[j0 + 1]], sc1,
                                     add=True)
                    return carry

                lax.fori_loop(0, NH // 2, body, 0)
                pltpu.make_async_copy(rows0, acc.at[didx.at[NH - 2]],
                                      sc0).wait()
                pltpu.make_async_copy(rows1, acc.at[didx.at[NH - 1]],
                                      sc1).wait()

            plsc.subcore_barrier()
            pltpu.sync_copy(acc.at[pl.ds(base, RPT)],
                            out_hbm.at[c, pl.ds(base, RPT)])

        pl.run_scoped(run,
                      pltpu.VMEM((NB // 2, B), jnp.int32),
                      pltpu.VMEM((NB // 2, B), jnp.int32),
                      pltpu.VMEM((B, D), jnp.float32),
                      pltpu.VMEM((B, D), jnp.float32))

    return agg


_agg128 = _make_agg(128)
_agg64 = _make_agg(64, tc_tiling=False)


_RB = 1024


def _dinv_of(degp_ref):
    d = degp_ref[0, :, 0] + degp_ref[1, :, 0] + 1.0
    return lax.rsqrt(d)[:, None]


def _tc_scale_body(degp_ref, x_ref, o_ref):
    o_ref[...] = x_ref[...] * _dinv_of(degp_ref)


def _tc_mid_body(degp_ref, p_ref, xs_ref, w1_ref, b1_ref, w2_ref, o_ref):
    dinv = _dinv_of(degp_ref)
    agg = (p_ref[0] + p_ref[1] + xs_ref[...]) * dinv
    h = jnp.maximum(jnp.dot(agg, w1_ref[...],
                            preferred_element_type=jnp.float32)
                    + b1_ref[...], 0.0)
    o_ref[...] = jnp.dot(h, w2_ref[...],
                         preferred_element_type=jnp.float32) * dinv


def _tc_dec_body(degp_ref, q_ref, xs_ref, b2_ref, w1_ref, b1_ref, w2_ref,
                 b2d_ref, o_ref):
    dinv = _dinv_of(degp_ref)
    z = (q_ref[0] + q_ref[1] + xs_ref[...]) * dinv + b2_ref[...]
    h = jnp.maximum(jnp.dot(z, w1_ref[...],
                            preferred_element_type=jnp.float32)
                    + b1_ref[...], 0.0)
    o_ref[...] = jnp.dot(h, w2_ref[...],
                         preferred_element_type=jnp.float32) + b2d_ref[...]


def _row_spec(d):
    return pl.BlockSpec((_RB, d), lambda i: (i, 0))


def _part_spec(d):
    return pl.BlockSpec((2, _RB, d), lambda i: (0, i, 0))


def _full_spec(a, b):
    return pl.BlockSpec((a, b), lambda i: (0, 0))


def _tc_scale(degp, x):
    return pl.pallas_call(
        _tc_scale_body,
        grid=(NP // _RB,),
        in_specs=[_part_spec(1), _row_spec(128)],
        out_specs=_row_spec(128),
        out_shape=jax.ShapeDtypeStruct((NP, 128), jnp.float32),
    )(degp, x)


def _tc_mid(degp, p, xs1, w1, b1, w2):
    return pl.pallas_call(
        _tc_mid_body,
        grid=(NP // _RB,),
        in_specs=[_part_spec(1), _part_spec(128), _row_spec(128),
                  _full_spec(128, 256), _full_spec(1, 256),
                  _full_spec(256, 64)],
        out_specs=_row_spec(64),
        out_shape=jax.ShapeDtypeStruct((NP, 64), jnp.float32),
    )(degp, p, xs1, w1, b1, w2)


def _tc_dec(degp, q, xs2, b2, w1, b1, w2, b2d):
    return pl.pallas_call(
        _tc_dec_body,
        grid=(NP // _RB,),
        in_specs=[_part_spec(1), _part_spec(64), _row_spec(64),
                  _full_spec(1, 64), _full_spec(64, 256), _full_spec(1, 256),
                  _full_spec(256, 8), _full_spec(1, 8)],
        out_specs=_row_spec(8),
        out_shape=jax.ShapeDtypeStruct((NP, 8), jnp.float32),
    )(degp, q, xs2, b2, w1, b1, w2, b2d)


def kernel(x, edge_index, conv1_W, conv1_b, conv2_W, conv2_b,
           fc1_W, fc1_b, fc2_W, fc2_b):
    ei = edge_index.astype(jnp.int32)
    src = ei[0].reshape(NT, NB, B)
    dst = ei[1].reshape(NT, NB, B)

    xpad = jnp.pad(x, ((0, NP - N), (0, 0)))

    degp = _deg(dst, jnp.ones((B,), jnp.float32),
                jnp.zeros((NP,), jnp.float32))[:, :, None]
    xs1 = _tc_scale(degp, xpad)
    p = _agg128(xs1, src, dst, jnp.zeros((NP, 128), jnp.float32))
    xs2 = _tc_mid(degp, p, xs1, conv1_W, conv1_b[None, :], conv2_W)
    q = _agg64(xs2, src, dst, jnp.zeros((NP, 64), jnp.float32))

    fc2p = jnp.pad(fc2_W[:, :6], ((0, 0), (0, 2)))
    fc2bp = jnp.pad(fc2_b[:6], (0, 2))
    out = _tc_dec(degp, q, xs2, conv2_b[None, :], fc1_W, fc1_b[None, :],
                  fc2p, fc2bp[None, :])
    return out[:N, :6]


# speedup vs baseline: 1.0444x; 1.0271x over previous
import functools

import jax
import jax.numpy as jnp
from jax import lax
from jax.experimental import pallas as pl
from jax.experimental.pallas import tpu as pltpu
from jax.experimental.pallas import tpu_sc as plsc

N = 10000
NP = 10240
NT = 32
NS = 16
EPT = 10000
B = 125
NB = EPT // B
RPT = NP // NS

_MESH = plsc.VectorSubcoreMesh(core_axis_name="c", subcore_axis_name="s")


@functools.partial(
    pl.kernel,
    out_type=jax.ShapeDtypeStruct((2, NP), jnp.float32),
    mesh=_MESH,
    scratch_types=[
        pltpu.VMEM((NB, B), jnp.int32),
        pltpu.VMEM((B,), jnp.float32),
        pltpu.VMEM_SHARED((NP,), jnp.float32),
        pltpu.SemaphoreType.DMA,
    ],
)
def _deg(dst_hbm, ones_hbm, zero_hbm, out_hbm, didx, ones_v, acc, dsem):
    c = lax.axis_index("c")
    s = lax.axis_index("s")
    wid = c * NS + s
    base = s * RPT

    pltpu.sync_copy(ones_hbm, ones_v)
    pltpu.sync_copy(dst_hbm.at[wid], didx)
    pltpu.sync_copy(zero_hbm.at[pl.ds(base, RPT)], acc.at[pl.ds(base, RPT)])
    plsc.subcore_barrier()

    def body(ch, carry):
        def fire(k, c2):
            pltpu.async_copy(ones_v, acc.at[didx.at[ch * 16 + k]], dsem,
                             add=True)
            return c2

        lax.fori_loop(0, 16, fire, 0)

        def drain(k, c2):
            pltpu.make_async_copy(ones_v, acc.at[didx.at[ch * 16 + k]],
                                  dsem).wait()
            return c2

        lax.fori_loop(0, 16, drain, 0)
        return carry

    lax.fori_loop(0, NB // 16, body, 0)
    plsc.subcore_barrier()
    pltpu.sync_copy(acc.at[pl.ds(base, RPT)], out_hbm.at[c, pl.ds(base, RPT)])


def _make_agg(D, tc_tiling=True):
    @functools.partial(
        pl.kernel,
        out_type=jax.ShapeDtypeStruct((2, NP, D), jnp.float32),
        mesh=_MESH,
        compiler_params=pltpu.CompilerParams(use_tc_tiling_on_sc=tc_tiling),
        scratch_types=[
            pltpu.VMEM_SHARED((NP, D), jnp.float32),
            pltpu.SemaphoreType.DMA,
            pltpu.SemaphoreType.DMA,
            pltpu.SemaphoreType.DMA,
            pltpu.SemaphoreType.DMA,
        ],
    )
    def agg(xs_hbm, src_hbm, dst_hbm, zero_hbm, out_hbm, acc, ga0, ga1,
            sc0, sc1):
        c = lax.axis_index("c")
        s = lax.axis_index("s")
        wid = c * NS + s
        base = s * RPT

        NH = NB // 2

        def run(sidx, didx, rows0, rows1):
            pltpu.sync_copy(zero_hbm.at[pl.ds(base, RPT)],
                            acc.at[pl.ds(base, RPT)])
            plsc.subcore_barrier()

            for p in range(2):
                pltpu.sync_copy(src_hbm.at[wid, pl.ds(p * NH, NH)], sidx)
                pltpu.sync_copy(dst_hbm.at[wid, pl.ds(p * NH, NH)], didx)
                pltpu.async_copy(xs_hbm.at[sidx.at[0]], rows0, ga0)

                def body(jj, carry):
                    j0 = 2 * jj

                    @pl.when(jj > 0)
                    def _():
                        pltpu.make_async_copy(
                            rows1, acc.at[didx.at[j0 - 1]], sc1).wait()

                    pltpu.async_copy(xs_hbm.at[sidx.at[j0 + 1]], rows1, ga1)
                    pltpu.make_async_copy(xs_hbm.at[sidx.at[j0]], rows0,
                                          ga0).wait()
                    pltpu.async_copy(rows0, acc.at[didx.at[j0]], sc0,
                                     add=True)

                    @pl.when(jj < NH // 2 - 1)
                    def _():
                        pltpu.make_async_copy(
                            rows0, acc.at[didx.at[j0]], sc0).wait()
                        pltpu.async_copy(xs_hbm.at[sidx.at[j0 + 2]], rows0,
                                         ga0)

                    pltpu.make_async_copy(xs_hbm.at[sidx.at[j0 + 1]], rows1,
                                          ga1).wait()
                    pltpu.async_copy(rows1, acc.at[didx.at[j0 + 1]], sc1,
                                     add=True)
                    return carry

                lax.fori_loop(0, NH // 2, body, 0)
                pltpu.make_async_copy(rows0, acc.at[didx.at[NH - 2]],
                                      sc0).wait()
                pltpu.make_async_copy(rows1, acc.at[didx.at[NH - 1]],
                                      sc1).wait()

            plsc.subcore_barrier()
            pltpu.sync_copy(acc.at[pl.ds(base, RPT)],
                            out_hbm.at[c, pl.ds(base, RPT)])

        pl.run_scoped(run,
                      pltpu.VMEM((NB // 2, B), jnp.int32),
                      pltpu.VMEM((NB // 2, B), jnp.int32),
                      pltpu.VMEM((B, D), jnp.float32),
                      pltpu.VMEM((B, D), jnp.float32))

    return agg


_agg128 = _make_agg(128)
_agg64 = _make_agg(64, tc_tiling=False)


_RB = 2048


def _dinv_of(degp_ref):
    d = degp_ref[0, :, 0] + degp_ref[1, :, 0] + 1.0
    return lax.rsqrt(d)[:, None]


def _tc_scale_body(degp_ref, x_ref, o_ref):
    o_ref[...] = x_ref[...] * _dinv_of(degp_ref)


def _tc_mid_body(degp_ref, p_ref, xs_ref, w1_ref, b1_ref, w2_ref, o_ref):
    dinv = _dinv_of(degp_ref)
    agg = (p_ref[0] + p_ref[1] + xs_ref[...]) * dinv
    h = jnp.maximum(jnp.dot(agg, w1_ref[...],
                            preferred_element_type=jnp.float32)
                    + b1_ref[...], 0.0)
    o_ref[...] = jnp.dot(h, w2_ref[...],
                         preferred_element_type=jnp.float32) * dinv


def _tc_dec_body(degp_ref, q_ref, xs_ref, b2_ref, w1_ref, b1_ref, w2_ref,
                 b2d_ref, o_ref):
    dinv = _dinv_of(degp_ref)
    z = (q_ref[0] + q_ref[1] + xs_ref[...]) * dinv + b2_ref[...]
    h = jnp.maximum(jnp.dot(z, w1_ref[...],
                            preferred_element_type=jnp.float32)
                    + b1_ref[...], 0.0)
    o_ref[...] = jnp.dot(h, w2_ref[...],
                         preferred_element_type=jnp.float32) + b2d_ref[...]


def _row_spec(d):
    return pl.BlockSpec((_RB, d), lambda i: (i, 0))


def _part_spec(d):
    return pl.BlockSpec((2, _RB, d), lambda i: (0, i, 0))


def _full_spec(a, b):
    return pl.BlockSpec((a, b), lambda i: (0, 0))


def _tc_scale(degp, x):
    return pl.pallas_call(
        _tc_scale_body,
        grid=(NP // _RB,),
        in_specs=[_part_spec(1), _row_spec(128)],
        out_specs=_row_spec(128),
        out_shape=jax.ShapeDtypeStruct((NP, 128), jnp.float32),
    )(degp, x)


def _tc_mid(degp, p, xs1, w1, b1, w2):
    return pl.pallas_call(
        _tc_mid_body,
        grid=(NP // _RB,),
        in_specs=[_part_spec(1), _part_spec(128), _row_spec(128),
                  _full_spec(128, 256), _full_spec(1, 256),
                  _full_spec(256, 64)],
        out_specs=_row_spec(64),
        out_shape=jax.ShapeDtypeStruct((NP, 64), jnp.float32),
    )(degp, p, xs1, w1, b1, w2)


def _tc_dec(degp, q, xs2, b2, w1, b1, w2, b2d):
    return pl.pallas_call(
        _tc_dec_body,
        grid=(NP // _RB,),
        in_specs=[_part_spec(1), _part_spec(64), _row_spec(64),
                  _full_spec(1, 64), _full_spec(64, 256), _full_spec(1, 256),
                  _full_spec(256, 8), _full_spec(1, 8)],
        out_specs=_row_spec(8),
        out_shape=jax.ShapeDtypeStruct((NP, 8), jnp.float32),
    )(degp, q, xs2, b2, w1, b1, w2, b2d)


def kernel(x, edge_index, conv1_W, conv1_b, conv2_W, conv2_b,
           fc1_W, fc1_b, fc2_W, fc2_b):
    ei = edge_index.astype(jnp.int32)
    src = ei[0].reshape(NT, NB, B)
    dst = ei[1].reshape(NT, NB, B)

    xpad = jnp.pad(x, ((0, NP - N), (0, 0)))

    degp = _deg(dst, jnp.ones((B,), jnp.float32),
                jnp.zeros((NP,), jnp.float32))[:, :, None]
    xs1 = _tc_scale(degp, xpad)
    p = _agg128(xs1, src, dst, jnp.zeros((NP, 128), jnp.float32))
    xs2 = _tc_mid(degp, p, xs1, conv1_W, conv1_b[None, :], conv2_W)
    q = _agg64(xs2, src, dst, jnp.zeros((NP, 64), jnp.float32))

    fc2p = jnp.pad(fc2_W[:, :6], ((0, 0), (0, 2)))
    fc2bp = jnp.pad(fc2_b[:6], (0, 2))
    out = _tc_dec(degp, q, xs2, conv2_b[None, :], fc1_W, fc1_b[None, :],
                  fc2p, fc2bp[None, :])
    return out[:N, :6]


# speedup vs baseline: 1.0474x; 1.0029x over previous
import functools

import jax
import jax.numpy as jnp
from jax import lax
from jax.experimental import pallas as pl
from jax.experimental.pallas import tpu as pltpu
from jax.experimental.pallas import tpu_sc as plsc

N = 10000
NP = 10240
NT = 32
NS = 16
EPT = 10000
B = 125
NB = EPT // B
RPT = NP // NS

_MESH = plsc.VectorSubcoreMesh(core_axis_name="c", subcore_axis_name="s")


@functools.partial(
    pl.kernel,
    out_type=jax.ShapeDtypeStruct((2, NP), jnp.float32),
    mesh=_MESH,
    scratch_types=[
        pltpu.VMEM((NB, B), jnp.int32),
        pltpu.VMEM((B,), jnp.float32),
        pltpu.VMEM_SHARED((NP,), jnp.float32),
        pltpu.SemaphoreType.DMA,
    ],
)
def _deg(dst_hbm, ones_hbm, zero_hbm, out_hbm, didx, ones_v, acc, dsem):
    c = lax.axis_index("c")
    s = lax.axis_index("s")
    wid = c * NS + s
    base = s * RPT

    pltpu.sync_copy(ones_hbm, ones_v)
    pltpu.sync_copy(dst_hbm.at[wid], didx)
    pltpu.sync_copy(zero_hbm.at[pl.ds(base, RPT)], acc.at[pl.ds(base, RPT)])
    plsc.subcore_barrier()

    def body(ch, carry):
        def fire(k, c2):
            pltpu.async_copy(ones_v, acc.at[didx.at[ch * 16 + k]], dsem,
                             add=True)
            return c2

        lax.fori_loop(0, 16, fire, 0)

        def drain(k, c2):
            pltpu.make_async_copy(ones_v, acc.at[didx.at[ch * 16 + k]],
                                  dsem).wait()
            return c2

        lax.fori_loop(0, 16, drain, 0)
        return carry

    lax.fori_loop(0, NB // 16, body, 0)
    plsc.subcore_barrier()
    pltpu.sync_copy(acc.at[pl.ds(base, RPT)], out_hbm.at[c, pl.ds(base, RPT)])


def _make_agg(D, tc_tiling=True):
    @functools.partial(
        pl.kernel,
        out_type=jax.ShapeDtypeStruct((2, NP, D), jnp.float32),
        mesh=_MESH,
        compiler_params=pltpu.CompilerParams(use_tc_tiling_on_sc=tc_tiling),
        scratch_types=[
            pltpu.VMEM_SHARED((NP, D), jnp.float32),
            pltpu.SemaphoreType.DMA,
            pltpu.SemaphoreType.DMA,
            pltpu.SemaphoreType.DMA,
            pltpu.SemaphoreType.DMA,
        ],
    )
    def agg(xs_hbm, src_hbm, dst_hbm, zero_hbm, out_hbm, acc, ga0, ga1,
            sc0, sc1):
        c = lax.axis_index("c")
        s = lax.axis_index("s")
        wid = c * NS + s
        base = s * RPT

        NH = NB // 2

        def run(sidx, didx, rows0, rows1):
            pltpu.sync_copy(zero_hbm.at[pl.ds(base, RPT)],
                            acc.at[pl.ds(base, RPT)])
            plsc.subcore_barrier()

            for p in range(2):
                pltpu.sync_copy(src_hbm.at[wid, pl.ds(p * NH, NH)], sidx)
                pltpu.sync_copy(dst_hbm.at[wid, pl.ds(p * NH, NH)], didx)
                pltpu.async_copy(xs_hbm.at[sidx.at[0]], rows0, ga0)

                def body(jj, carry):
                    j0 = 2 * jj

                    @pl.when(jj > 0)
                    def _():
                        pltpu.make_async_copy(
                            rows1, acc.at[didx.at[j0 - 1]], sc1).wait()

                    pltpu.async_copy(xs_hbm.at[sidx.at[j0 + 1]], rows1, ga1)
                    pltpu.make_async_copy(xs_hbm.at[sidx.at[j0]], rows0,
                                          ga0).wait()
                    pltpu.async_copy(rows0, acc.at[didx.at[j0]], sc0,
                                     add=True)

                    @pl.when(jj < NH // 2 - 1)
                    def _():
                        pltpu.make_async_copy(
                            rows0, acc.at[didx.at[j0]], sc0).wait()
                        pltpu.async_copy(xs_hbm.at[sidx.at[j0 + 2]], rows0,
                                         ga0)

                    pltpu.make_async_copy(xs_hbm.at[sidx.at[j0 + 1]], rows1,
                                          ga1).wait()
                    pltpu.async_copy(rows1, acc.at[didx.at[j0 + 1]], sc1,
                                     add=True)
                    return carry

                lax.fori_loop(0, NH // 2, body, 0)
                pltpu.make_async_copy(rows0, acc.at[didx.at[NH - 2]],
                                      sc0).wait()
                pltpu.make_async_copy(rows1, acc.at[didx.at[NH - 1]],
                                      sc1).wait()

            plsc.subcore_barrier()
            pltpu.sync_copy(acc.at[pl.ds(base, RPT)],
                            out_hbm.at[c, pl.ds(base, RPT)])

        pl.run_scoped(run,
                      pltpu.VMEM((NB // 2, B), jnp.int32),
                      pltpu.VMEM((NB // 2, B), jnp.int32),
                      pltpu.VMEM((B, D), jnp.float32),
                      pltpu.VMEM((B, D), jnp.float32))

    return agg


_agg128 = _make_agg(128)
_agg64 = _make_agg(64, tc_tiling=False)


_RB = 2560


def _dinv_of(degp_ref):
    d = degp_ref[0, :, 0] + degp_ref[1, :, 0] + 1.0
    return lax.rsqrt(d)[:, None]


def _tc_scale_body(degp_ref, x_ref, o_ref):
    o_ref[...] = x_ref[...] * _dinv_of(degp_ref)


def _tc_mid_body(degp_ref, p_ref, xs_ref, w1_ref, b1_ref, w2_ref, o_ref):
    dinv = _dinv_of(degp_ref)
    agg = (p_ref[0] + p_ref[1] + xs_ref[...]) * dinv
    h = jnp.maximum(jnp.dot(agg, w1_ref[...],
                            preferred_element_type=jnp.float32)
                    + b1_ref[...], 0.0)
    o_ref[...] = jnp.dot(h, w2_ref[...],
                         preferred_element_type=jnp.float32) * dinv


def _tc_dec_body(degp_ref, q_ref, xs_ref, b2_ref, w1_ref, b1_ref, w2_ref,
                 b2d_ref, o_ref):
    dinv = _dinv_of(degp_ref)
    z = (q_ref[0] + q_ref[1] + xs_ref[...]) * dinv + b2_ref[...]
    h = jnp.maximum(jnp.dot(z, w1_ref[...],
                            preferred_element_type=jnp.float32)
                    + b1_ref[...], 0.0)
    o_ref[...] = jnp.dot(h, w2_ref[...],
                         preferred_element_type=jnp.float32) + b2d_ref[...]


def _row_spec(d):
    return pl.BlockSpec((_RB, d), lambda i: (i, 0))


def _part_spec(d):
    return pl.BlockSpec((2, _RB, d), lambda i: (0, i, 0))


def _full_spec(a, b):
    return pl.BlockSpec((a, b), lambda i: (0, 0))


def _tc_scale(degp, x):
    return pl.pallas_call(
        _tc_scale_body,
        grid=(NP // _RB,),
        in_specs=[_part_spec(1), _row_spec(128)],
        out_specs=_row_spec(128),
        out_shape=jax.ShapeDtypeStruct((NP, 128), jnp.float32),
    )(degp, x)


def _tc_mid(degp, p, xs1, w1, b1, w2):
    return pl.pallas_call(
        _tc_mid_body,
        grid=(NP // _RB,),
        in_specs=[_part_spec(1), _part_spec(128), _row_spec(128),
                  _full_spec(128, 256), _full_spec(1, 256),
                  _full_spec(256, 64)],
        out_specs=_row_spec(64),
        out_shape=jax.ShapeDtypeStruct((NP, 64), jnp.float32),
    )(degp, p, xs1, w1, b1, w2)


def _tc_dec(degp, q, xs2, b2, w1, b1, w2, b2d):
    return pl.pallas_call(
        _tc_dec_body,
        grid=(NP // _RB,),
        in_specs=[_part_spec(1), _part_spec(64), _row_spec(64),
                  _full_spec(1, 64), _full_spec(64, 256), _full_spec(1, 256),
                  _full_spec(256, 8), _full_spec(1, 8)],
        out_specs=_row_spec(8),
        out_shape=jax.ShapeDtypeStruct((NP, 8), jnp.float32),
    )(degp, q, xs2, b2, w1, b1, w2, b2d)


def kernel(x, edge_index, conv1_W, conv1_b, conv2_W, conv2_b,
           fc1_W, fc1_b, fc2_W, fc2_b):
    ei = edge_index.astype(jnp.int32)
    src = ei[0].reshape(NT, NB, B)
    dst = ei[1].reshape(NT, NB, B)

    xpad = jnp.pad(x, ((0, NP - N), (0, 0)))

    degp = _deg(dst, jnp.ones((B,), jnp.float32),
                jnp.zeros((NP,), jnp.float32))[:, :, None]
    xs1 = _tc_scale(degp, xpad)
    p = _agg128(xs1, src, dst, jnp.zeros((NP, 128), jnp.float32))
    xs2 = _tc_mid(degp, p, xs1, conv1_W, conv1_b[None, :], conv2_W)
    q = _agg64(xs2, src, dst, jnp.zeros((NP, 64), jnp.float32))

    fc2p = jnp.pad(fc2_W[:, :6], ((0, 0), (0, 2)))
    fc2bp = jnp.pad(fc2_b[:6], (0, 2))
    out = _tc_dec(degp, q, xs2, conv2_b[None, :], fc1_W, fc1_b[None, :],
                  fc2p, fc2bp[None, :])
    return out[:N, :6]
